# baseline (speedup 1.0000x reference)
import jax
import jax.numpy as jnp
from jax.experimental import pallas as pl
from jax.experimental.pallas import tpu as pltpu

N_INPUTS = 8192
N_COLUMNS = 4096
K = 64
CONNECTED_PERM = 0.2
BETA = 3.0
_CB = 256
_NBLK = N_COLUMNS // _CB


def _stage1_body(x_ref, duty_ref, perm_ref, bw_ref, out_ref):
    connb = (perm_ref[...] >= CONNECTED_PERM).astype(jnp.float32)
    ov = jnp.dot(connb, x_ref[...].reshape(N_INPUTS, 1),
                 preferred_element_type=jnp.float32).reshape(1, _CB)
    sm = jnp.dot(bw_ref[...], duty_ref[...].reshape(N_COLUMNS, 1),
                 preferred_element_type=jnp.float32).reshape(1, _CB)
    boost = jnp.exp(BETA * (K / N_COLUMNS - sm))
    out_ref[...] = ov * boost


_TR = 8
_TC = N_COLUMNS // _TR


def _topk_body(v_ref, idx_ref):
    vals0 = v_ref[...]
    r_io = jax.lax.broadcasted_iota(jnp.int32, (_TR, _TC), 0)
    c_io = jax.lax.broadcasted_iota(jnp.int32, (_TR, _TC), 1)
    gid = r_io * _TC + c_io
    lanek = jax.lax.broadcasted_iota(jnp.int32, (1, K), 1)

    def step(j, carry):
        vals, out = carry
        m = jnp.max(vals)
        idx = jnp.min(jnp.where(vals == m, gid, N_COLUMNS))
        out = jnp.where(lanek == j, idx, out)
        vals = jnp.where(gid == idx, -jnp.inf, vals)
        return vals, out

    _, out = jax.lax.fori_loop(
        0, K, step, (vals0, jnp.zeros((1, K), jnp.int32)))
    idx_ref[...] = out


def kernel(x, permanences, potential_mask_f, duty_cycle, boost_weights):
    del potential_mask_f
    boosted = pl.pallas_call(
        _stage1_body,
        grid=(_NBLK,),
        in_specs=[
            pl.BlockSpec((1, N_INPUTS), lambda i: (0, 0)),
            pl.BlockSpec((1, N_COLUMNS), lambda i: (0, 0)),
            pl.BlockSpec((_CB, N_INPUTS), lambda i: (i, 0)),
            pl.BlockSpec((_CB, N_COLUMNS), lambda i: (i, 0)),
        ],
        out_specs=pl.BlockSpec((1, _CB), lambda i: (0, i)),
        out_shape=jax.ShapeDtypeStruct((1, N_COLUMNS), jnp.float32),
    )(x.reshape(1, N_INPUTS), duty_cycle.reshape(1, N_COLUMNS),
      permanences, boost_weights)
    idx = pl.pallas_call(
        _topk_body,
        out_shape=jax.ShapeDtypeStruct((1, K), jnp.int32),
    )(boosted.reshape(_TR, _TC))
    return idx.reshape(K)
